# Initial kernel scaffold; baseline (speedup 1.0000x reference)
import functools

import jax
import jax.numpy as jnp
from jax import lax
from jax.experimental import pallas as pl
from jax.experimental.pallas import tpu as pltpu
from jax.experimental.pallas import tpu_sc as plsc

N = 10000
E = 320000
D = 128
M = 32
AVG_DEG = 32.0

NC = 2
NS = 16
NW = NC * NS
CH = 128
K = -(-E // (NW * CH))
E_PAD = NW * K * CH
TW = 48
N_ACC = 10016
RPT = N_ACC // NS

_mesh = plsc.VectorSubcoreMesh(
    core_axis_name="c", subcore_axis_name="s", num_cores=NC, num_subcores=NS)


@functools.partial(
    pl.kernel,
    out_type=[jax.ShapeDtypeStruct((E_PAD, TW), jnp.float32),
              jax.ShapeDtypeStruct((E_PAD, TW), jnp.float32)],
    mesh=_mesh,
    scratch_types=[
        pltpu.VMEM((K, CH), jnp.int32),
        pltpu.VMEM((K, CH), jnp.int32),
        pltpu.VMEM((CH, TW), jnp.float32),
        pltpu.VMEM((CH, TW), jnp.float32),
        pltpu.SemaphoreType.DMA,
        pltpu.SemaphoreType.DMA,
    ],
)
def _sc_gather(a_hbm, b_hbm, srcs_hbm, dsts_hbm, ga_hbm, gb_hbm,
               idx_a, idx_b, abuf, bbuf, sema, semb):
    c = lax.axis_index("c")
    s = lax.axis_index("s")
    wid = s * NC + c
    base = wid * (K * CH)
    pltpu.sync_copy(srcs_hbm.at[wid], idx_a)
    pltpu.sync_copy(dsts_hbm.at[wid], idx_b)

    @pl.loop(0, K)
    def _chunk(cc):
        off = base + cc * CH
        da = pltpu.async_copy(a_hbm.at[idx_a.at[cc]], abuf, sema)
        db = pltpu.async_copy(b_hbm.at[idx_b.at[cc]], bbuf, semb)
        da.wait()
        db.wait()
        pltpu.sync_copy(abuf, ga_hbm.at[pl.ds(off, CH)])
        pltpu.sync_copy(bbuf, gb_hbm.at[pl.ds(off, CH)])


@functools.partial(
    pl.kernel,
    out_type=jax.ShapeDtypeStruct((NC * N_ACC, TW), jnp.float32),
    mesh=_mesh,
    scratch_types=[
        pltpu.VMEM((K, CH), jnp.int32),
        pltpu.VMEM((CH, TW), jnp.float32),
        pltpu.VMEM_SHARED((N_ACC, TW), jnp.float32),
    ],
)
def _sc_scatter(s_hbm, dsts_hbm, z_hbm, out_hbm, idx, sbuf, accum):
    c = lax.axis_index("c")
    s = lax.axis_index("s")
    wid = s * NC + c
    base = wid * (K * CH)
    pltpu.sync_copy(z_hbm.at[pl.ds(s * RPT, RPT)], accum.at[pl.ds(s * RPT, RPT)])
    pltpu.sync_copy(dsts_hbm.at[wid], idx)
    plsc.subcore_barrier()

    @pl.loop(0, K)
    def _chunk(cc):
        pltpu.sync_copy(s_hbm.at[pl.ds(base + cc * CH, CH)], sbuf)
        pltpu.sync_copy(sbuf, accum.at[idx.at[cc]], add=True)

    plsc.subcore_barrier()
    pltpu.sync_copy(accum.at[pl.ds(s * RPT, RPT)],
                    out_hbm.at[pl.ds(c * N_ACC + s * RPT, RPT)])


def _tables_body(h_ref, c_ref, w1a_ref, w1b_ref, b1_ref, a_ref, b_ref):
    h = h_ref[...]
    pa = jnp.dot(h, w1a_ref[...], preferred_element_type=jnp.float32) + b1_ref[...]
    pb = jnp.dot(h, w1b_ref[...], preferred_element_type=jnp.float32)
    coords = c_ref[...]
    pad = jnp.zeros((h.shape[0], TW - M - 3), jnp.float32)
    a_ref[...] = jnp.concatenate([pa, coords, pad], axis=1)
    b_ref[...] = jnp.concatenate([pb, coords, pad], axis=1)


def _edge_body(w1c_ref, we2_ref, be2_ref, wc1_ref, bc1_ref, wc2_ref,
               ga_ref, gb_ref, s_ref):
    ga = ga_ref[...]
    gb = gb_ref[...]
    rel = ga[:, M:M + 3] - gb[:, M:M + 3]
    d2 = jnp.sum(rel * rel, axis=1, keepdims=True)
    pre = ga[:, :M] + gb[:, :M] + d2 * w1c_ref[...]
    m = jax.nn.silu(pre)
    m = jax.nn.silu(jnp.dot(m, we2_ref[...], preferred_element_type=jnp.float32)
                    + be2_ref[...])
    t = jax.nn.silu(jnp.dot(m, wc1_ref[...], preferred_element_type=jnp.float32)
                    + bc1_ref[...])
    cw = jnp.tanh(jnp.dot(t, wc2_ref[...], preferred_element_type=jnp.float32))
    pad = jnp.zeros((ga.shape[0], TW - M - 3), jnp.float32)
    s_ref[...] = jnp.concatenate([m, rel * cw, pad], axis=1)


def _node_body(c_ref, h_ref, parts_ref, wn1a_ref, wn1b_ref, bn1_ref,
               wn2_ref, bn2_ref, oc_ref, oh_ref):
    parts = parts_ref[...]
    agg = parts[:N, :] + parts[N_ACC:N_ACC + N, :]
    agg_m = agg[:, :M]
    agg_c = agg[:, M:M + 3]
    oc_ref[...] = c_ref[...] + agg_c * (1.0 / AVG_DEG)
    h = h_ref[...]
    u = jax.nn.silu(
        jnp.dot(h, wn1a_ref[...], preferred_element_type=jnp.float32)
        + jnp.dot(agg_m, wn1b_ref[...], preferred_element_type=jnp.float32)
        + bn1_ref[...])
    oh = h + jnp.dot(u, wn2_ref[...], preferred_element_type=jnp.float32) + bn2_ref[...]
    hc = oh - jnp.mean(oh, axis=0, keepdims=True)
    denom = jnp.sqrt(jnp.mean(jnp.sum(hc * hc, axis=1)) + 1e-6)
    oh_ref[...] = hc / denom


def kernel(coords, hidden, edges, W_e1, b_e1, W_e2, b_e2, W_c1, b_c1, W_c2,
           W_n1, b_n1, W_n2, b_n2):
    src = edges[0].astype(jnp.int32)
    dst = edges[1].astype(jnp.int32)
    pad = E_PAD - E
    src_g = jnp.concatenate([src, jnp.zeros((pad,), jnp.int32)]).reshape(NW, K, CH)
    dst_g = jnp.concatenate([dst, jnp.zeros((pad,), jnp.int32)]).reshape(NW, K, CH)
    dst_s = jnp.concatenate([dst, jnp.full((pad,), N, jnp.int32)]).reshape(NW, K, CH)

    w1a = W_e1[:D]
    w1b = W_e1[D:2 * D]
    w1c = W_e1[2 * D].reshape(1, M)

    tab_a, tab_b = pl.pallas_call(
        _tables_body,
        out_shape=[jax.ShapeDtypeStruct((N, TW), jnp.float32),
                   jax.ShapeDtypeStruct((N, TW), jnp.float32)],
    )(hidden, coords, w1a, w1b, b_e1.reshape(1, M))

    ga, gb = _sc_gather(tab_a, tab_b, src_g, dst_g)

    BE = 2048
    n_blk = E_PAD // BE
    s_rows = pl.pallas_call(
        _edge_body,
        grid=(n_blk,),
        in_specs=[
            pl.BlockSpec((1, M), lambda i: (0, 0)),
            pl.BlockSpec((M, M), lambda i: (0, 0)),
            pl.BlockSpec((1, M), lambda i: (0, 0)),
            pl.BlockSpec((M, M), lambda i: (0, 0)),
            pl.BlockSpec((1, M), lambda i: (0, 0)),
            pl.BlockSpec((M, 1), lambda i: (0, 0)),
            pl.BlockSpec((BE, TW), lambda i: (i, 0)),
            pl.BlockSpec((BE, TW), lambda i: (i, 0)),
        ],
        out_specs=pl.BlockSpec((BE, TW), lambda i: (i, 0)),
        out_shape=jax.ShapeDtypeStruct((E_PAD, TW), jnp.float32),
    )(w1c, W_e2, b_e2.reshape(1, M), W_c1, b_c1.reshape(1, M), W_c2, ga, gb)

    zeros_acc = jnp.zeros((N_ACC, TW), jnp.float32)
    parts = _sc_scatter(s_rows, dst_s, zeros_acc)

    out_coords, out_hidden = pl.pallas_call(
        _node_body,
        out_shape=[jax.ShapeDtypeStruct((N, 3), jnp.float32),
                   jax.ShapeDtypeStruct((N, D), jnp.float32)],
    )(coords, hidden, parts, W_n1[:D], W_n1[D:], b_n1.reshape(1, D),
      W_n2, b_n2.reshape(1, D))

    return out_coords, out_hidden


# speedup vs baseline: 4.9469x; 4.9469x over previous
import functools

import jax
import jax.numpy as jnp
from jax import lax
from jax.experimental import pallas as pl
from jax.experimental.pallas import tpu as pltpu
from jax.experimental.pallas import tpu_sc as plsc

N = 10000
E = 320000
D = 128
M = 32
AVG_DEG = 32.0

NC = 2
NS = 16
NW = NC * NS
CH = 128
K = -(-E // (NW * CH))
E_PAD = NW * K * CH
TW = 48
N_ACC = 10112
RPT = N_ACC // NS

_mesh = plsc.VectorSubcoreMesh(
    core_axis_name="c", subcore_axis_name="s", num_cores=NC, num_subcores=NS)


@functools.partial(
    pl.kernel,
    out_type=[jax.ShapeDtypeStruct((E_PAD, TW), jnp.float32),
              jax.ShapeDtypeStruct((E_PAD, TW), jnp.float32)],
    mesh=_mesh,
    scratch_types=[
        pltpu.VMEM((K, CH), jnp.int32),
        pltpu.VMEM((K, CH), jnp.int32),
        pltpu.VMEM((CH, TW), jnp.float32),
        pltpu.VMEM((CH, TW), jnp.float32),
        pltpu.SemaphoreType.DMA,
        pltpu.SemaphoreType.DMA,
    ],
    compiler_params=pltpu.CompilerParams(use_tc_tiling_on_sc=False),
)
def _sc_gather(a_hbm, b_hbm, srcs_hbm, dsts_hbm, ga_hbm, gb_hbm,
               idx_a, idx_b, abuf, bbuf, sema, semb):
    c = lax.axis_index("c")
    s = lax.axis_index("s")
    wid = s * NC + c
    base = wid * (K * CH)
    pltpu.sync_copy(srcs_hbm.at[wid], idx_a)
    pltpu.sync_copy(dsts_hbm.at[wid], idx_b)

    @pl.loop(0, K)
    def _chunk(cc):
        off = base + cc * CH
        da = pltpu.async_copy(a_hbm.at[idx_a.at[cc]], abuf, sema)
        db = pltpu.async_copy(b_hbm.at[idx_b.at[cc]], bbuf, semb)
        da.wait()
        db.wait()
        pltpu.sync_copy(abuf, ga_hbm.at[pl.ds(off, CH)])
        pltpu.sync_copy(bbuf, gb_hbm.at[pl.ds(off, CH)])


@functools.partial(
    pl.kernel,
    out_type=jax.ShapeDtypeStruct((NC * N_ACC, TW), jnp.float32),
    mesh=_mesh,
    scratch_types=[
        pltpu.VMEM((K, CH), jnp.int32),
        pltpu.VMEM((CH, TW), jnp.float32),
        pltpu.VMEM_SHARED((N_ACC, TW), jnp.float32),
    ],
    compiler_params=pltpu.CompilerParams(use_tc_tiling_on_sc=False),
)
def _sc_scatter(s_hbm, dsts_hbm, z_hbm, out_hbm, idx, sbuf, accum):
    c = lax.axis_index("c")
    s = lax.axis_index("s")
    wid = s * NC + c
    base = wid * (K * CH)
    pltpu.sync_copy(z_hbm.at[pl.ds(s * RPT, RPT)], accum.at[pl.ds(s * RPT, RPT)])
    pltpu.sync_copy(dsts_hbm.at[wid], idx)
    plsc.subcore_barrier()

    @pl.loop(0, K)
    def _chunk(cc):
        pltpu.sync_copy(s_hbm.at[pl.ds(base + cc * CH, CH)], sbuf)
        pltpu.sync_copy(sbuf, accum.at[idx.at[cc]], add=True)

    plsc.subcore_barrier()
    pltpu.sync_copy(accum.at[pl.ds(s * RPT, RPT)],
                    out_hbm.at[pl.ds(c * N_ACC + s * RPT, RPT)])


def _tables_body(h_ref, c_ref, w1a_ref, w1b_ref, b1_ref, a_ref, b_ref):
    h = h_ref[...]
    pa = jnp.dot(h, w1a_ref[...], preferred_element_type=jnp.float32) + b1_ref[...]
    pb = jnp.dot(h, w1b_ref[...], preferred_element_type=jnp.float32)
    coords = c_ref[...]
    pad = jnp.zeros((h.shape[0], TW - M - 3), jnp.float32)
    a_ref[...] = jnp.concatenate([pa, coords, pad], axis=1)
    b_ref[...] = jnp.concatenate([pb, coords, pad], axis=1)


def _edge_body(w1c_ref, we2_ref, be2_ref, wc1_ref, bc1_ref, wc2_ref,
               ga_ref, gb_ref, s_ref):
    ga = ga_ref[...]
    gb = gb_ref[...]
    rel = ga[:, M:M + 3] - gb[:, M:M + 3]
    d2 = jnp.sum(rel * rel, axis=1, keepdims=True)
    pre = ga[:, :M] + gb[:, :M] + d2 * w1c_ref[...]
    m = jax.nn.silu(pre)
    m = jax.nn.silu(jnp.dot(m, we2_ref[...], preferred_element_type=jnp.float32)
                    + be2_ref[...])
    t = jax.nn.silu(jnp.dot(m, wc1_ref[...], preferred_element_type=jnp.float32)
                    + bc1_ref[...])
    cw = jnp.tanh(jnp.dot(t, wc2_ref[...], preferred_element_type=jnp.float32))
    pad = jnp.zeros((ga.shape[0], TW - M - 3), jnp.float32)
    s_ref[...] = jnp.concatenate([m, rel * cw, pad], axis=1)


def _node_body(c_ref, h_ref, parts_ref, wn1a_ref, wn1b_ref, bn1_ref,
               wn2_ref, bn2_ref, oc_ref, oh_ref):
    parts = parts_ref[...]
    agg = parts[:N, :] + parts[N_ACC:N_ACC + N, :]
    agg_m = agg[:, :M]
    agg_c = agg[:, M:M + 3]
    oc_ref[...] = c_ref[...] + agg_c * (1.0 / AVG_DEG)
    h = h_ref[...]
    u = jax.nn.silu(
        jnp.dot(h, wn1a_ref[...], preferred_element_type=jnp.float32)
        + jnp.dot(agg_m, wn1b_ref[...], preferred_element_type=jnp.float32)
        + bn1_ref[...])
    oh = h + jnp.dot(u, wn2_ref[...], preferred_element_type=jnp.float32) + bn2_ref[...]
    hc = oh - jnp.mean(oh, axis=0, keepdims=True)
    denom = jnp.sqrt(jnp.mean(jnp.sum(hc * hc, axis=1)) + 1e-6)
    oh_ref[...] = hc / denom


def kernel(coords, hidden, edges, W_e1, b_e1, W_e2, b_e2, W_c1, b_c1, W_c2,
           W_n1, b_n1, W_n2, b_n2):
    src = edges[0].astype(jnp.int32)
    dst = edges[1].astype(jnp.int32)
    pad = E_PAD - E
    src_g = jnp.concatenate([src, jnp.zeros((pad,), jnp.int32)]).reshape(NW, K, CH)
    dst_g = jnp.concatenate([dst, jnp.zeros((pad,), jnp.int32)]).reshape(NW, K, CH)
    dst_s = jnp.concatenate([dst, jnp.full((pad,), N, jnp.int32)]).reshape(NW, K, CH)

    w1a = W_e1[:D]
    w1b = W_e1[D:2 * D]
    w1c = W_e1[2 * D].reshape(1, M)

    tab_a, tab_b = pl.pallas_call(
        _tables_body,
        out_shape=[jax.ShapeDtypeStruct((N, TW), jnp.float32),
                   jax.ShapeDtypeStruct((N, TW), jnp.float32)],
    )(hidden, coords, w1a, w1b, b_e1.reshape(1, M))

    ga, gb = _sc_gather(tab_a, tab_b, src_g, dst_g)

    BE = 2048
    n_blk = E_PAD // BE
    s_rows = pl.pallas_call(
        _edge_body,
        grid=(n_blk,),
        in_specs=[
            pl.BlockSpec((1, M), lambda i: (0, 0)),
            pl.BlockSpec((M, M), lambda i: (0, 0)),
            pl.BlockSpec((1, M), lambda i: (0, 0)),
            pl.BlockSpec((M, M), lambda i: (0, 0)),
            pl.BlockSpec((1, M), lambda i: (0, 0)),
            pl.BlockSpec((M, 1), lambda i: (0, 0)),
            pl.BlockSpec((BE, TW), lambda i: (i, 0)),
            pl.BlockSpec((BE, TW), lambda i: (i, 0)),
        ],
        out_specs=pl.BlockSpec((BE, TW), lambda i: (i, 0)),
        out_shape=jax.ShapeDtypeStruct((E_PAD, TW), jnp.float32),
    )(w1c, W_e2, b_e2.reshape(1, M), W_c1, b_c1.reshape(1, M), W_c2, ga, gb)

    zeros_acc = jnp.zeros((N_ACC, TW), jnp.float32)
    parts = _sc_scatter(s_rows, dst_s, zeros_acc)

    out_coords, out_hidden = pl.pallas_call(
        _node_body,
        out_shape=[jax.ShapeDtypeStruct((N, 3), jnp.float32),
                   jax.ShapeDtypeStruct((N, D), jnp.float32)],
    )(coords, hidden, parts, W_n1[:D], W_n1[D:], b_n1.reshape(1, D),
      W_n2, b_n2.reshape(1, D))

    return out_coords, out_hidden
